# baseline (speedup 1.0000x reference)
import jax
import jax.numpy as jnp
import numpy as np
from jax.experimental import pallas as pl
from jax.experimental.pallas import tpu as pltpu

_LAMBDA_0, _MU_0 = 1.0, 0.5
_KAPPA_LAMBDA, _KAPPA_MU = 2.0, 1.0
_BASE_WEIGHT, _GRADIENT_SCALING = 1.0, 5.0
_CLAMP_MIN, _CLAMP_MAX = 0.1, 10.0
_SCALE_WEIGHTS = (1.0, 0.5, 0.25)
_JAC_PENALTY_W = 0.1
_BLUR_SIGMA = 1.1

_tt = np.arange(5, dtype=np.float64) - 2.0
_kk = np.exp(-(_tt ** 2) / (2.0 * _BLUR_SIGMA ** 2))
_BLUR_TAPS = tuple(float(v) for v in (_kk / _kk.sum()).astype(np.float32))


def _interp_matrix(n, out_size):
    coords = jnp.arange(out_size, dtype=jnp.float32) * ((n - 1) / max(out_size - 1, 1))
    i0 = jnp.floor(coords).astype(jnp.int32)
    i1 = jnp.minimum(i0 + 1, n - 1)
    w = coords - i0.astype(jnp.float32)
    cols = jnp.arange(n, dtype=jnp.int32)
    r = ((1.0 - w)[:, None] * (cols[None, :] == i0[:, None])
         + w[:, None] * (cols[None, :] == i1[:, None]))
    return r.astype(jnp.float32)


def _resize3d(x, out_sizes):
    hp = jax.lax.Precision.HIGHEST
    n3, n2, n1 = x.shape[-3], x.shape[-2], x.shape[-1]
    s3, s2, s1 = out_sizes
    if (n3, n2, n1) == (s3, s2, s1):
        return x
    x = jnp.einsum('...xyz,ox->...oyz', x, _interp_matrix(n3, s3), precision=hp)
    x = jnp.einsum('...xyz,oy->...xoz', x, _interp_matrix(n2, s2), precision=hp)
    x = jnp.einsum('...xyz,oz->...xyo', x, _interp_matrix(n1, s1), precision=hp)
    return x


def _grad_y(f):
    return jnp.concatenate([
        f[..., 1:2, :] - f[..., 0:1, :],
        0.5 * (f[..., 2:, :] - f[..., :-2, :]),
        f[..., -1:, :] - f[..., -2:-1, :],
    ], axis=-2)


def _grad_z(f):
    return jnp.concatenate([
        f[..., 1:2] - f[..., 0:1],
        0.5 * (f[..., 2:] - f[..., :-2]),
        f[..., -1:] - f[..., -2:-1],
    ], axis=-1)


def _ig_body(main_ref, left_ref, right_ref, out_ref, *, nx):
    i = pl.program_id(1)
    first, last = i == 0, i == nx - 1
    m = main_ref[0]
    lrow = jnp.where(first, 2.0 * m[0:1] - m[1:2], left_ref[0][1:2])
    rrow = jnp.where(last, 2.0 * m[-1:] - m[-2:-1], right_ref[0][0:1])
    q = jnp.concatenate([lrow, m, rrow], axis=0)
    gx = 0.5 * (q[2:] - q[:-2])
    gy = _grad_y(m)
    gz = _grad_z(m)
    out_ref[0] = jnp.sqrt(gx * gx + gy * gy + gz * gz)


def _energy_body(dmain_ref, dleft_ref, dright_ref, gmain_ref, gleft_ref,
                 gright_ref, out_ref, *, nx, t):
    i = pl.program_id(1)
    first, last = i == 0, i == nx - 1

    m = gmain_ref[0]
    lpart = jnp.where(first, jnp.concatenate([m[2:3], m[1:2]], axis=0),
                      gleft_ref[0][2:4])
    rpart = jnp.where(last, jnp.concatenate([m[-2:-1], m[-3:-2]], axis=0),
                      gright_ref[0][0:2])
    gq = jnp.concatenate([lpart, m, rpart], axis=0)
    taps = _BLUR_TAPS
    n = m.shape[-1]
    gze = jnp.concatenate([gq[:, :, 2:3], gq[:, :, 1:2], gq,
                           gq[:, :, -2:-1], gq[:, :, -3:-2]], axis=2)
    bz = sum(taps[k] * gze[:, :, k:k + n] for k in range(5))
    bye = jnp.concatenate([bz[:, 2:3], bz[:, 1:2], bz,
                           bz[:, -2:-1], bz[:, -3:-2]], axis=1)
    by = sum(taps[k] * bye[:, k:k + n, :] for k in range(5))
    ig = sum(taps[k] * by[k:k + t] for k in range(5))

    lam = jnp.clip(_LAMBDA_0 + _KAPPA_LAMBDA * ig, _CLAMP_MIN, _CLAMP_MAX)
    mu = jnp.clip(_MU_0 + _KAPPA_MU * ig, _CLAMP_MIN, _CLAMP_MAX)
    wgt = _BASE_WEIGHT + _GRADIENT_SCALING * ig

    dm = dmain_ref[0]
    ld = jnp.where(first, 2.0 * dm[:, 0:1] - dm[:, 1:2], dleft_ref[0][:, 1:2])
    rd = jnp.where(last, 2.0 * dm[:, -1:] - dm[:, -2:-1], dright_ref[0][:, 0:1])
    dq = jnp.concatenate([ld, dm, rd], axis=1)

    def gx(c):
        return 0.5 * (dq[c, 2:] - dq[c, :-2])

    e_xx, e_yy, e_zz = gx(0), _grad_y(dm[1]), _grad_z(dm[2])
    e_xy = 0.5 * (_grad_y(dm[0]) + gx(1))
    e_xz = 0.5 * (_grad_z(dm[0]) + gx(2))
    e_yz = 0.5 * (_grad_z(dm[1]) + _grad_y(dm[2]))
    tr = e_xx + e_yy + e_zz
    energy = (0.5 * lam * tr * tr
              + mu * (e_xx * e_xx + e_yy * e_yy + e_zz * e_zz
                      + 2.0 * (e_xy * e_xy + e_xz * e_xz + e_yz * e_yz)))
    s = jnp.sum(wgt * energy)
    out_ref[...] = jnp.broadcast_to(s.reshape(1, 1, 1, 1), (1, 1, 8, 128))


def _scale_loss(deform_s, image_s):
    b = deform_s.shape[0]
    n = deform_s.shape[-1]
    t = 8 if n % 8 == 0 else 4
    nx = n // t
    h2, h4 = n // 2 - 1, n // 4 - 1

    def lmap2(bb, i):
        return (bb, jnp.maximum(i * t // 2 - 1, 0), 0, 0)

    def rmap2(bb, i):
        return (bb, jnp.minimum((i + 1) * t // 2, h2), 0, 0)

    ig = pl.pallas_call(
        lambda *refs: _ig_body(*refs, nx=nx),
        grid=(b, nx),
        in_specs=[
            pl.BlockSpec((1, t, n, n), lambda bb, i: (bb, i, 0, 0)),
            pl.BlockSpec((1, 2, n, n), lmap2),
            pl.BlockSpec((1, 2, n, n), rmap2),
        ],
        out_specs=pl.BlockSpec((1, t, n, n), lambda bb, i: (bb, i, 0, 0)),
        out_shape=jax.ShapeDtypeStruct((b, n, n, n), jnp.float32),
        compiler_params=pltpu.CompilerParams(
            dimension_semantics=("parallel", "parallel")),
    )(image_s, image_s, image_s)

    partials = pl.pallas_call(
        lambda *refs: _energy_body(*refs, nx=nx, t=t),
        grid=(b, nx),
        in_specs=[
            pl.BlockSpec((1, 3, t, n, n), lambda bb, i: (bb, 0, i, 0, 0)),
            pl.BlockSpec((1, 3, 2, n, n),
                         lambda bb, i: (bb, 0, jnp.maximum(i * t // 2 - 1, 0), 0, 0)),
            pl.BlockSpec((1, 3, 2, n, n),
                         lambda bb, i: (bb, 0, jnp.minimum((i + 1) * t // 2, h2), 0, 0)),
            pl.BlockSpec((1, t, n, n), lambda bb, i: (bb, i, 0, 0)),
            pl.BlockSpec((1, 4, n, n),
                         lambda bb, i: (bb, jnp.maximum(i * t // 4 - 1, 0), 0, 0)),
            pl.BlockSpec((1, 4, n, n),
                         lambda bb, i: (bb, jnp.minimum((i + 1) * t // 4, h4), 0, 0)),
        ],
        out_specs=pl.BlockSpec((1, 1, 8, 128), lambda bb, i: (bb, i, 0, 0)),
        out_shape=jax.ShapeDtypeStruct((b, nx, 8, 128), jnp.float32),
        compiler_params=pltpu.CompilerParams(
            dimension_semantics=("parallel", "parallel")),
    )(deform_s, deform_s, deform_s, ig, ig, ig)

    return jnp.sum(partials[:, :, 0, 0]) / (b * n * n * n)


def _jacobian_penalty(deform):
    b, _, x, y, z = deform.shape
    c = (x // 2, y // 2, z // 2)
    dx = 0.5 * (deform[:, :, c[0] + 1, c[1], c[2]] - deform[:, :, c[0] - 1, c[1], c[2]])
    dy = 0.5 * (deform[:, :, c[0], c[1] + 1, c[2]] - deform[:, :, c[0], c[1] - 1, c[2]])
    dz = 0.5 * (deform[:, :, c[0], c[1], c[2] + 1] - deform[:, :, c[0], c[1], c[2] - 1])
    jac = jnp.stack([dx, dy, dz], axis=-1)
    det = jnp.linalg.det(jac)
    return jnp.mean(jax.nn.relu(-det))


def kernel(deformation_field, image):
    bsz, _, x, y, z = deformation_field.shape
    total = jnp.zeros((), dtype=deformation_field.dtype)
    for i, sw in enumerate(_SCALE_WEIGHTS):
        scale = 2 ** i
        out_sizes = (x // scale, y // scale, z // scale)
        deform_s = _resize3d(deformation_field, out_sizes)
        image_s = _resize3d(image, out_sizes)[:, 0]
        total = total + sw * _scale_loss(deform_s, image_s)
    return total + _JAC_PENALTY_W * _jacobian_penalty(deformation_field)
